# baseline (speedup 1.0000x reference)
import functools

import jax
import jax.numpy as jnp
from jax import lax
from jax.experimental import pallas as pl
from jax.experimental.pallas import tpu as pltpu
from jax.experimental.pallas import tpu_sc as plsc

EMB_DIM = 64
BATCH, SEQ = 16384, 50
NUM_CORES = 2
NUM_SUBCORES = 16
NW = NUM_CORES * NUM_SUBCORES
CHUNK = 128
NCB = BATCH // CHUNK
CB_PER_W = NCB // NW
RT = EMB_DIM // 8
JB = 2
NBATCH = SEQ * (CB_PER_W // JB)


def _make_gather():
    mesh = plsc.VectorSubcoreMesh(core_axis_name="c", subcore_axis_name="s")

    @functools.partial(
        pl.kernel,
        mesh=mesh,
        out_type=jax.ShapeDtypeStruct((SEQ, RT, NCB, 8, CHUNK), jnp.float32),
        scratch_types=[
            pltpu.VMEM((2, JB, CHUNK), jnp.int32),
            pltpu.VMEM((2, JB, CHUNK, EMB_DIM), jnp.float32),
            pltpu.VMEM((2, RT, JB, 8, CHUNK), jnp.float32),
            pltpu.SemaphoreType.DMA((2,)),
            pltpu.SemaphoreType.DMA((2,)),
        ],
        compiler_params=pltpu.CompilerParams(use_tc_tiling_on_sc=False,
                                             needs_layout_passes=False),
    )
    def gather_kernel(idx_hbm, table_hbm, out_hbm, idx_v, rows_v, tbuf,
                      gsem, wsem):
        wid = lax.axis_index("s") * NUM_CORES + lax.axis_index("c")
        lane = jax.lax.iota(jnp.int32, 16)

        def fetch_and_fire(s, cb0, bb):
            for j in range(JB):
                pltpu.sync_copy(idx_hbm.at[s // 8, cb0 + j, s % 8],
                                idx_v.at[bb, j])
                pltpu.async_copy(table_hbm.at[idx_v.at[bb, j]],
                                 rows_v.at[bb, j], gsem.at[bb])

        def wait_gathers(bb):
            for j in range(JB):
                pltpu.make_async_copy(table_hbm.at[idx_v.at[bb, j]],
                                      rows_v.at[bb, j], gsem.at[bb]).wait()

        def wait_writes(bb):
            for r in range(RT):
                pltpu.make_async_copy(tbuf.at[bb, r],
                                      out_hbm.at[0, r, pl.ds(0, JB)],
                                      wsem.at[bb]).wait()

        def transpose(bb):
            def tbody(r, carry):
                for k in range(8):
                    col = jnp.full((16,), r * 8 + k, jnp.int32)
                    for j in range(JB):
                        src = rows_v.at[bb, j]
                        for lg in range(8):
                            v = plsc.load_gather(src, [lane + lg * 16, col])
                            tbuf[bb, r, j, k, pl.ds(lg * 16, 16)] = v
                return carry

            lax.fori_loop(0, RT, tbody, 0, unroll=False)

        def fire_writes(s, cb0, bb):
            for r in range(RT):
                pltpu.async_copy(tbuf.at[bb, r],
                                 out_hbm.at[s, r, pl.ds(cb0, JB)],
                                 wsem.at[bb])

        cb_base = wid * CB_PER_W
        fetch_and_fire(0, cb_base, 0)

        def body(i, carry):
            for p in range(2):
                bb = p
                cb0 = cb_base + p * JB
                s_nxt = i + p
                cb_nxt = cb_base + (1 - p) * JB

                @pl.when(jnp.logical_or(i < SEQ - 1, p < 1))
                def _pref():
                    fetch_and_fire(s_nxt, cb_nxt, 1 - bb)

                wait_gathers(bb)

                @pl.when(i >= 1)
                def _drain():
                    wait_writes(bb)

                transpose(bb)
                fire_writes(i, cb0, bb)
            return carry

        lax.fori_loop(0, SEQ, body, 0, unroll=False)
        for bb in range(2):
            wait_writes(bb)

    return gather_kernel


_gather = _make_gather()


def kernel(input, weight):
    iv = jnp.pad(input.T, ((0, 56 - SEQ), (0, 0)))
    iv = iv.reshape(7, 8, NCB, CHUNK).transpose(0, 2, 1, 3)
    out5 = _gather(iv, weight)
    return out5.transpose(2, 4, 0, 1, 3).reshape(BATCH, SEQ, EMB_DIM)


# speedup vs baseline: 1.1472x; 1.1472x over previous
import functools

import jax
import jax.numpy as jnp
from jax import lax
from jax.experimental import pallas as pl
from jax.experimental.pallas import tpu as pltpu
from jax.experimental.pallas import tpu_sc as plsc

EMB_DIM = 64
BATCH, SEQ = 16384, 50
NUM_CORES = 2
NUM_SUBCORES = 16
NW = NUM_CORES * NUM_SUBCORES
CHUNK = 128
NCB = BATCH // CHUNK
CB_PER_W = NCB // NW
RT = EMB_DIM // 8
JB = 2
NBATCH = SEQ * (CB_PER_W // JB)


def _make_gather():
    mesh = plsc.VectorSubcoreMesh(core_axis_name="c", subcore_axis_name="s")

    @functools.partial(
        pl.kernel,
        mesh=mesh,
        out_type=jax.ShapeDtypeStruct((SEQ, RT, NCB, 8, CHUNK), jnp.float32),
        scratch_types=[
            pltpu.VMEM((2, JB, CHUNK), jnp.int32),
            pltpu.VMEM((2, JB, CHUNK, EMB_DIM), jnp.float32),
            pltpu.VMEM((2, RT, JB, 8, CHUNK), jnp.float32),
            pltpu.SemaphoreType.DMA((2,)),
            pltpu.SemaphoreType.DMA((2,)),
        ],
        compiler_params=pltpu.CompilerParams(use_tc_tiling_on_sc=False,
                                             needs_layout_passes=False),
    )
    def gather_kernel(idx_hbm, table_hbm, out_hbm, idx_v, rows_v, tbuf,
                      gsem, wsem):
        wid = lax.axis_index("s") * NUM_CORES + lax.axis_index("c")
        lane = jax.lax.iota(jnp.int32, 16)

        def fetch_and_fire(s, cb0, bb):
            for j in range(JB):
                pltpu.sync_copy(idx_hbm.at[s // 8, cb0 + j, s % 8],
                                idx_v.at[bb, j])
                pltpu.async_copy(table_hbm.at[idx_v.at[bb, j]],
                                 rows_v.at[bb, j], gsem.at[bb])

        def wait_gathers(bb):
            for j in range(JB):
                pltpu.make_async_copy(table_hbm.at[idx_v.at[bb, j]],
                                      rows_v.at[bb, j], gsem.at[bb]).wait()

        def wait_writes(bb):
            for r in range(RT):
                pltpu.make_async_copy(tbuf.at[bb, r],
                                      out_hbm.at[0, r, pl.ds(0, JB)],
                                      wsem.at[bb]).wait()

        def transpose(bb):
            def tbody(r, carry):
                for k in range(8):
                    col = jnp.full((16,), r * 8 + k, jnp.int32)
                    for j in range(JB):
                        src = rows_v.at[bb, j]
                        vs = [plsc.load_gather(src, [lane + lg * 16, col])
                              for lg in range(8)]
                        for lg in range(8):
                            tbuf[bb, r, j, k, pl.ds(lg * 16, 16)] = vs[lg]
                return carry

            lax.fori_loop(0, RT, tbody, 0, unroll=False)

        def fire_writes(s, cb0, bb):
            for r in range(RT):
                pltpu.async_copy(tbuf.at[bb, r],
                                 out_hbm.at[s, r, pl.ds(cb0, JB)],
                                 wsem.at[bb])

        cb_base = wid * CB_PER_W
        fetch_and_fire(0, cb_base, 0)

        def body(i, carry):
            for p in range(2):
                bb = p
                cb0 = cb_base + p * JB
                s_nxt = i + p
                cb_nxt = cb_base + (1 - p) * JB

                @pl.when(jnp.logical_or(i < SEQ - 1, p < 1))
                def _pref():
                    fetch_and_fire(s_nxt, cb_nxt, 1 - bb)

                wait_gathers(bb)

                @pl.when(i >= 1)
                def _drain():
                    wait_writes(bb)

                transpose(bb)
                fire_writes(i, cb0, bb)
            return carry

        lax.fori_loop(0, SEQ, body, 0, unroll=False)
        for bb in range(2):
            wait_writes(bb)

    return gather_kernel


_gather = _make_gather()


def kernel(input, weight):
    iv = jnp.pad(input.T, ((0, 56 - SEQ), (0, 0)))
    iv = iv.reshape(7, 8, NCB, CHUNK).transpose(0, 2, 1, 3)
    out5 = _gather(iv, weight)
    return out5.transpose(2, 4, 0, 1, 3).reshape(BATCH, SEQ, EMB_DIM)


# speedup vs baseline: 1.7189x; 1.4982x over previous
import functools

import jax
import jax.numpy as jnp
from jax import lax
from jax.experimental import pallas as pl
from jax.experimental.pallas import tpu as pltpu
from jax.experimental.pallas import tpu_sc as plsc

EMB_DIM = 64
BATCH, SEQ = 16384, 50
NUM_CORES = 2
NUM_SUBCORES = 16
NW = NUM_CORES * NUM_SUBCORES
CHUNK = 128
NCB = BATCH // CHUNK
CB_PER_W = NCB // NW
RT = EMB_DIM // 8
JB = 2
PITCH = 129
TROWS = JB * RT * 8


def _make_gather():
    mesh = plsc.VectorSubcoreMesh(core_axis_name="c", subcore_axis_name="s")

    @functools.partial(
        pl.kernel,
        mesh=mesh,
        out_type=jax.ShapeDtypeStruct((SEQ, RT, NCB, 8, CHUNK), jnp.float32),
        scratch_types=[
            pltpu.VMEM((2, JB, CHUNK), jnp.int32),
            pltpu.VMEM((2, JB, CHUNK, EMB_DIM), jnp.float32),
            pltpu.VMEM((2, TROWS, PITCH), jnp.float32),
            pltpu.SemaphoreType.DMA((2,)),
            pltpu.SemaphoreType.DMA((2,)),
        ],
        compiler_params=pltpu.CompilerParams(use_tc_tiling_on_sc=False,
                                             needs_layout_passes=False),
    )
    def gather_kernel(idx_hbm, table_hbm, out_hbm, idx_v, rows_v, tbuf,
                      gsem, wsem):
        wid = lax.axis_index("s") * NUM_CORES + lax.axis_index("c")
        lane = jax.lax.iota(jnp.int32, 16)
        rowid = [[jnp.full((16,), j * 64 + g * 16, jnp.int32) + lane
                  for g in range(4)] for j in range(JB)]

        def fetch_and_fire(s, cb0, bb):
            for j in range(JB):
                pltpu.sync_copy(idx_hbm.at[s // 8, cb0 + j, s % 8],
                                idx_v.at[bb, j])
                pltpu.async_copy(table_hbm.at[idx_v.at[bb, j]],
                                 rows_v.at[bb, j], gsem.at[bb])

        def wait_gathers(bb):
            for j in range(JB):
                pltpu.make_async_copy(table_hbm.at[idx_v.at[bb, j]],
                                      rows_v.at[bb, j], gsem.at[bb]).wait()

        def wait_writes(bb):
            for j in range(JB):
                for r in range(RT):
                    pltpu.make_async_copy(
                        tbuf.at[bb, pl.ds(0, 8), pl.ds(0, CHUNK)],
                        out_hbm.at[0, r, 0], wsem.at[bb]).wait()

        def transpose(bb):
            def tbody(lq, carry):
                for lu in range(4):
                    ll = lq * 4 + lu
                    l_full = jnp.full((16,), 0, jnp.int32) + ll
                    for j in range(JB):
                        src = rows_v.at[bb, j]
                        for g in range(4):
                            v = src[ll, pl.ds(g * 16, 16)]
                            plsc.store_scatter(tbuf.at[bb],
                                               [rowid[j][g], l_full], v)
                return carry

            lax.fori_loop(0, CHUNK // 4, tbody, 0, unroll=False)

        def fire_writes(s, cb0, bb):
            for j in range(JB):
                for r in range(RT):
                    pltpu.async_copy(
                        tbuf.at[bb, pl.ds(j * 64 + r * 8, 8), pl.ds(0, CHUNK)],
                        out_hbm.at[s, r, cb0 + j], wsem.at[bb])

        cb_base = wid * CB_PER_W
        fetch_and_fire(0, cb_base, 0)

        def body(i, carry):
            for p in range(2):
                bb = p
                cb0 = cb_base + p * JB
                s_nxt = i + p
                cb_nxt = cb_base + (1 - p) * JB

                @pl.when(jnp.logical_or(i < SEQ - 1, p < 1))
                def _pref():
                    fetch_and_fire(s_nxt, cb_nxt, 1 - bb)

                wait_gathers(bb)

                @pl.when(i >= 1)
                def _drain():
                    wait_writes(bb)

                transpose(bb)
                fire_writes(i, cb0, bb)
            return carry

        lax.fori_loop(0, SEQ, body, 0, unroll=False)
        for bb in range(2):
            wait_writes(bb)

    return gather_kernel


_gather = _make_gather()


def kernel(input, weight):
    iv = jnp.pad(input.T, ((0, 56 - SEQ), (0, 0)))
    iv = iv.reshape(7, 8, NCB, CHUNK).transpose(0, 2, 1, 3)
    out5 = _gather(iv, weight)
    return out5.transpose(2, 4, 0, 1, 3).reshape(BATCH, SEQ, EMB_DIM)


# speedup vs baseline: 1.9120x; 1.1124x over previous
import functools

import jax
import jax.numpy as jnp
from jax import lax
from jax.experimental import pallas as pl
from jax.experimental.pallas import tpu as pltpu
from jax.experimental.pallas import tpu_sc as plsc

EMB_DIM = 64
BATCH, SEQ = 16384, 50
NUM_CORES = 2
NUM_SUBCORES = 16
NW = NUM_CORES * NUM_SUBCORES
CHUNK = 128
NCB = BATCH // CHUNK
CB_PER_W = NCB // NW
RT = EMB_DIM // 8
JB = 2
PITCH = 129
TROWS = JB * RT * 8


def _make_gather():
    mesh = plsc.VectorSubcoreMesh(core_axis_name="c", subcore_axis_name="s")

    @functools.partial(
        pl.kernel,
        mesh=mesh,
        out_type=jax.ShapeDtypeStruct((SEQ, RT, NCB, 8, CHUNK), jnp.float32),
        scratch_types=[
            pltpu.VMEM((7, CB_PER_W, 8, CHUNK), jnp.int32),
            pltpu.VMEM((2, JB, CHUNK, EMB_DIM), jnp.float32),
            pltpu.VMEM((2, TROWS, PITCH), jnp.float32),
            pltpu.SemaphoreType.DMA((2,)),
            pltpu.SemaphoreType.DMA((2,)),
        ],
        compiler_params=pltpu.CompilerParams(use_tc_tiling_on_sc=False,
                                             needs_layout_passes=False),
    )
    def gather_kernel(idx_hbm, table_hbm, out_hbm, idx_v, rows_v, tbuf,
                      gsem, wsem):
        wid = lax.axis_index("s") * NUM_CORES + lax.axis_index("c")
        lane = jax.lax.iota(jnp.int32, 16)
        rowid = [[jnp.full((16,), j * 64 + g * 16, jnp.int32) + lane
                  for g in range(4)] for j in range(JB)]

        def fetch_and_fire(s, jbase, bb):
            for j in range(JB):
                pltpu.async_copy(
                    table_hbm.at[idx_v.at[s // 8, jbase + j, s % 8]],
                    rows_v.at[bb, j], gsem.at[bb])

        def wait_gathers(bb):
            for j in range(JB):
                pltpu.make_async_copy(table_hbm.at[idx_v.at[0, 0, 0]],
                                      rows_v.at[bb, j], gsem.at[bb]).wait()

        def wait_writes(bb):
            for j in range(JB):
                for r in range(RT):
                    pltpu.make_async_copy(
                        tbuf.at[bb, pl.ds(0, 8), pl.ds(0, CHUNK)],
                        out_hbm.at[0, r, 0], wsem.at[bb]).wait()

        def transpose(bb):
            def tbody(lq, carry):
                for lu in range(4):
                    ll = lq * 4 + lu
                    l_full = jnp.full((16,), 0, jnp.int32) + ll
                    for j in range(JB):
                        src = rows_v.at[bb, j]
                        for g in range(4):
                            v = src[ll, pl.ds(g * 16, 16)]
                            plsc.store_scatter(tbuf.at[bb],
                                               [rowid[j][g], l_full], v)
                return carry

            lax.fori_loop(0, CHUNK // 4, tbody, 0, unroll=False)

        def fire_writes(s, cb0, bb):
            for j in range(JB):
                for r in range(RT):
                    pltpu.async_copy(
                        tbuf.at[bb, pl.ds(j * 64 + r * 8, 8), pl.ds(0, CHUNK)],
                        out_hbm.at[s, r, cb0 + j], wsem.at[bb])

        cb_base = wid * CB_PER_W
        pltpu.sync_copy(idx_hbm.at[:, pl.ds(cb_base, CB_PER_W)], idx_v)
        fetch_and_fire(0, 0, 0)

        def body(i, carry):
            for p in range(2):
                bb = p
                cb0 = cb_base + p * JB
                s_nxt = i + p
                jb_nxt = (1 - p) * JB

                @pl.when(jnp.logical_or(i < SEQ - 1, p < 1))
                def _pref():
                    fetch_and_fire(s_nxt, jb_nxt, 1 - bb)

                wait_gathers(bb)

                @pl.when(i >= 1)
                def _drain():
                    wait_writes(bb)

                transpose(bb)
                fire_writes(i, cb0, bb)
            return carry

        lax.fori_loop(0, SEQ, body, 0, unroll=False)
        for bb in range(2):
            wait_writes(bb)

    return gather_kernel


_gather = _make_gather()


def kernel(input, weight):
    iv = jnp.pad(input.T, ((0, 56 - SEQ), (0, 0)))
    iv = iv.reshape(7, 8, NCB, CHUNK).transpose(0, 2, 1, 3)
    out5 = _gather(iv, weight)
    return out5.transpose(2, 4, 0, 1, 3).reshape(BATCH, SEQ, EMB_DIM)
